# baseline (speedup 1.0000x reference)
import functools

import jax
import jax.numpy as jnp
from jax import lax
from jax.experimental import pallas as pl
from jax.experimental.pallas import tpu as pltpu
from jax.experimental.pallas import tpu_sc as plsc

_B = 16384
_D = 64
_NW = 32
_ROWS = _B // _NW
_LANES = 16
_CPD = _D // _LANES
_ICH = 64
_NICH = _ROWS // _ICH
_HALF = 250000


def _make_sc_kernel():
    mesh = plsc.VectorSubcoreMesh(core_axis_name="c", subcore_axis_name="s")

    @functools.partial(
        pl.kernel,
        mesh=mesh,
        out_type=jax.ShapeDtypeStruct((_NW * _LANES,), jnp.float32),
        scratch_types=[
            pltpu.VMEM((_NICH, _ICH), jnp.int32),
            pltpu.VMEM((_NICH, _ICH), jnp.int32),
            pltpu.VMEM((2, _ICH, 2 * _D), jnp.float32),
            pltpu.VMEM((2, _ICH, 2 * _D), jnp.float32),
            pltpu.VMEM((_ROWS // 2, 2 * _D), jnp.float32),
            pltpu.VMEM((_ROWS // 2, 2 * _D), jnp.float32),
            pltpu.VMEM((_LANES,), jnp.float32),
            pltpu.SemaphoreType.DMA,
            pltpu.SemaphoreType.DMA,
            pltpu.SemaphoreType.DMA,
        ],
    )
    def sc_kernel(ta_hbm, tb_hbm, ia_hbm, ib_hbm, w_hbm, feats_hbm, out_hbm,
                  ia_v, ib_v, ra_v, rb_v, w_v, feats_v, acc_v,
                  sem_g0, sem_g1, sem_l):
        wid = lax.axis_index("s") * 2 + lax.axis_index("c")
        sems_g = (sem_g0, sem_g1)

        fcp = pltpu.async_copy(
            feats_hbm.at[pl.ds(wid * (_ROWS // 2), _ROWS // 2)],
            feats_v, sem_l)
        wcp = pltpu.async_copy(
            w_hbm.at[pl.ds(wid * (_ROWS // 2), _ROWS // 2)],
            w_v, sem_l)
        pltpu.sync_copy(ia_hbm.at[pl.ds(wid * _NICH, _NICH)], ia_v)
        pltpu.sync_copy(ib_hbm.at[pl.ds(wid * _NICH, _NICH)], ib_v)

        def fire(ch):
            p = ch % 2
            a = pltpu.async_copy(
                ta_hbm.at[ia_v.at[ch]], ra_v.at[p], sems_g[p])
            b = pltpu.async_copy(
                tb_hbm.at[ib_v.at[ch]], rb_v.at[p], sems_g[p])
            return (a, b)

        gathers = [fire(0)]
        fcp.wait()
        wcp.wait()

        zero = jnp.zeros((_LANES,), jnp.float32)
        accs = (zero,) * _CPD
        for ch in range(_NICH):
            if ch + 1 < _NICH:
                gathers.append(fire(ch + 1))
            gathers[ch][0].wait()
            gathers[ch][1].wait()
            p = ch % 2

            def body(j, accs, p=p, ch=ch):
                wrow = ch * (_ICH // 2) + j
                out = list(accs)
                for h in range(2):
                    i = 2 * j + h
                    ws = [w_v[wrow, pl.ds(h * _D + q * _LANES, _LANES)]
                          for q in range(4)]
                    for c in range(_CPD):
                        cs = pl.ds(c * _LANES, _LANES)
                        cs_hi = pl.ds(_D + c * _LANES, _LANES)
                        sel = (ws[0] * ra_v[p, i, cs]
                               + ws[1] * ra_v[p, i, cs_hi]
                               + ws[2] * rb_v[p, i, cs]
                               + ws[3] * rb_v[p, i, cs_hi])
                        f = feats_v[wrow, pl.ds(h * _D + c * _LANES,
                                                _LANES)]
                        d = f - sel
                        out[c] = out[c] + d * d
                return tuple(out)

            accs = lax.fori_loop(0, _ICH // 2, body, accs)

        acc_v[...] = (accs[0] + accs[1]) + (accs[2] + accs[3])
        pltpu.sync_copy(acc_v, out_hbm.at[pl.ds(wid * _LANES, _LANES)])

    return sc_kernel


_SC_KERNEL = None


def kernel(feats, labels, centers):
    global _SC_KERNEL
    if _SC_KERNEL is None:
        _SC_KERNEL = _make_sc_kernel()
    labels32 = labels.astype(jnp.int32)
    n_half = centers.shape[0] // 2
    table_a = centers[:n_half].reshape(_HALF, 2 * _D)
    table_b = centers[n_half:].reshape(_HALF, 2 * _D)
    idx = labels32 >> 1
    in_a = idx < _HALF
    idx_a = jnp.where(in_a, idx, 0).reshape(_NW * _NICH, _ICH)
    idx_b = jnp.where(in_a, 0, idx - _HALF).reshape(_NW * _NICH, _ICH)
    lo = (labels32 & 1) == 0
    wf = jnp.stack(
        [in_a & lo, in_a & ~lo, ~in_a & lo, ~in_a & ~lo],
        axis=1).astype(jnp.float32)
    w = jnp.broadcast_to(wf[:, :, None], (_B, 4, _LANES))
    w = w.reshape(_B // 2, 2 * _D)
    feats128 = feats.reshape(_B // 2, 2 * _D)
    partials = _SC_KERNEL(table_a, table_b, idx_a, idx_b, w, feats128)
    return jnp.sum(partials) * (0.5 / _B)


# speedup vs baseline: 2.0753x; 2.0753x over previous
import functools

import jax
import jax.numpy as jnp
from jax import lax
from jax.experimental import pallas as pl
from jax.experimental.pallas import tpu as pltpu
from jax.experimental.pallas import tpu_sc as plsc

_B = 16384
_D = 64
_NW = 32
_ROWS = _B // _NW
_LANES = 16
_CPD = _D // _LANES
_ICH = 128
_NICH = _ROWS // _ICH


def _make_sc_kernel():
    mesh = plsc.VectorSubcoreMesh(core_axis_name="c", subcore_axis_name="s")

    @functools.partial(
        pl.kernel,
        mesh=mesh,
        out_type=jax.ShapeDtypeStruct((_NW * _LANES,), jnp.float32),
        scratch_types=[
            pltpu.VMEM((_NICH, _ICH), jnp.int32),
            pltpu.VMEM((2, _ICH, _D), jnp.float32),
            pltpu.VMEM((_ROWS // 2, 2 * _D), jnp.float32),
            pltpu.VMEM((_LANES,), jnp.float32),
            pltpu.SemaphoreType.DMA,
            pltpu.SemaphoreType.DMA,
            pltpu.SemaphoreType.DMA,
        ],
        compiler_params=pltpu.CompilerParams(use_tc_tiling_on_sc=False),
    )
    def sc_kernel(table_hbm, idx_hbm, feats_hbm, out_hbm,
                  idx_v, rows_v, feats_v, acc_v,
                  sem_g0, sem_g1, sem_l):
        wid = lax.axis_index("s") * 2 + lax.axis_index("c")
        sems_g = (sem_g0, sem_g1)

        fcp = pltpu.async_copy(
            feats_hbm.at[pl.ds(wid * (_ROWS // 2), _ROWS // 2)],
            feats_v, sem_l)
        pltpu.sync_copy(idx_hbm.at[pl.ds(wid * _NICH, _NICH)], idx_v)

        def fire(ch):
            p = ch % 2
            return pltpu.async_copy(
                table_hbm.at[idx_v.at[ch]], rows_v.at[p], sems_g[p])

        gathers = [fire(0)]
        fcp.wait()

        zero = jnp.zeros((_LANES,), jnp.float32)
        accs = (zero,) * _CPD
        for ch in range(_NICH):
            if ch + 1 < _NICH:
                gathers.append(fire(ch + 1))
            gathers[ch].wait()
            p = ch % 2

            def body(j, accs, p=p, ch=ch):
                frow = ch * (_ICH // 2) + j
                out = list(accs)
                for h in range(2):
                    i = 2 * j + h
                    for c in range(_CPD):
                        f = feats_v[frow,
                                    pl.ds(h * _D + c * _LANES, _LANES)]
                        g = rows_v[p, i, pl.ds(c * _LANES, _LANES)]
                        d = f - g
                        out[c] = out[c] + d * d
                return tuple(out)

            accs = lax.fori_loop(0, _ICH // 2, body, accs)

        acc_v[...] = (accs[0] + accs[1]) + (accs[2] + accs[3])
        pltpu.sync_copy(acc_v, out_hbm.at[pl.ds(wid * _LANES, _LANES)])

    return sc_kernel


_SC_KERNEL = None


def kernel(feats, labels, centers):
    global _SC_KERNEL
    if _SC_KERNEL is None:
        _SC_KERNEL = _make_sc_kernel()
    labels32 = labels.astype(jnp.int32)
    idx = labels32.reshape(_NW * _NICH, _ICH)
    feats128 = feats.reshape(_B // 2, 2 * _D)
    partials = _SC_KERNEL(centers, idx, feats128)
    return jnp.sum(partials) * (0.5 / _B)
